# Initial kernel scaffold; baseline (speedup 1.0000x reference)
import jax
import jax.numpy as jnp
from jax.experimental import pallas as pl


def kernel(input_ids_0, input_ids_1, attention_type_ids, relative_position_ids, table_0, table_1, attn_type_table, inverse_freqs):
    raise NotImplementedError("write your pallas kernel here")


# speedup vs baseline: 1.8501x; 1.8501x over previous
import functools

import jax
import jax.numpy as jnp
from jax import lax
from jax.experimental import pallas as pl
from jax.experimental.pallas import tpu as pltpu
from jax.experimental.pallas import tpu_sc as plsc

B, L = 1024, 50
HIDDEN = 768
HEAD_DIM = 64
NUM_ATT_TYPES = 8
BL = B * L

NC, NS = 2, 16
NW = NC * NS
ROWS_PER_W = BL // NW
CHUNK = 80
NCHUNK = ROWS_PER_W // CHUNK
LANES = 16
_VECS_PER_ROW = HIDDEN // LANES


def _emb_body(ids0_hbm, ids1_hbm, t0_hbm, t1_hbm, out_hbm,
              idx0_v, idx1_v, buf0, buf1, sem0, sem1):
    c = lax.axis_index("c")
    s = lax.axis_index("s")
    wid = s * NC + c
    pltpu.sync_copy(ids0_hbm.at[wid], idx0_v)
    pltpu.sync_copy(ids1_hbm.at[wid], idx1_v)
    row0 = wid * ROWS_PER_W

    def chunk_body(k, carry):
        cp0 = pltpu.async_copy(t0_hbm.at[idx0_v.at[k]], buf0, sem0)
        cp1 = pltpu.async_copy(t1_hbm.at[idx1_v.at[k]], buf1, sem1)
        cp0.wait()
        cp1.wait()

        def row_body(r, rc):
            for j in range(_VECS_PER_ROW):
                plsc.addupdate(buf0.at[r, pl.ds(j * LANES, LANES)],
                               buf1[r, pl.ds(j * LANES, LANES)])
            return rc

        lax.fori_loop(0, CHUNK, row_body, 0, unroll=False)
        pltpu.sync_copy(buf0, out_hbm.at[pl.ds(row0 + k * CHUNK, CHUNK)])
        return carry

    lax.fori_loop(0, NCHUNK, chunk_body, 0, unroll=False)


_emb_kernel = functools.partial(
    pl.kernel,
    out_type=jax.ShapeDtypeStruct((BL, HIDDEN), jnp.float32),
    mesh=plsc.VectorSubcoreMesh(core_axis_name="c", subcore_axis_name="s"),
    scratch_types=[
        pltpu.VMEM((NCHUNK, CHUNK), jnp.int32),
        pltpu.VMEM((NCHUNK, CHUNK), jnp.int32),
        pltpu.VMEM((CHUNK, HIDDEN), jnp.float32),
        pltpu.VMEM((CHUNK, HIDDEN), jnp.float32),
        pltpu.SemaphoreType.DMA,
        pltpu.SemaphoreType.DMA,
    ],
)(_emb_body)


_ROWS_BLK = 512
_NBLK = BL // _ROWS_BLK


def _pos_body(rel_ref, att_ref, freq_ref, phase_ref, table_ref, out_ref):
    rel = rel_ref[...].reshape(_ROWS_BLK, 1).astype(jnp.float32)
    x = rel / freq_ref[...] + phase_ref[...]
    posenc = jnp.sin(x)
    att = att_ref[...].reshape(_ROWS_BLK, 1)
    iota = lax.broadcasted_iota(jnp.int32, (_ROWS_BLK, NUM_ATT_TYPES), 1)
    onehot = (att == iota).astype(jnp.float32)
    att_emb = jnp.dot(onehot, table_ref[...],
                      preferred_element_type=jnp.float32)
    out_ref[...] = posenc + att_emb


def _pos_kernel(rel_ids, att_ids, freq2, phase, table):
    return pl.pallas_call(
        _pos_body,
        grid=(_NBLK,),
        in_specs=[
            pl.BlockSpec((1, _ROWS_BLK, 1), lambda i: (i, 0, 0)),
            pl.BlockSpec((1, _ROWS_BLK, 1), lambda i: (i, 0, 0)),
            pl.BlockSpec((1, HEAD_DIM), lambda i: (0, 0)),
            pl.BlockSpec((1, HEAD_DIM), lambda i: (0, 0)),
            pl.BlockSpec((NUM_ATT_TYPES, HEAD_DIM), lambda i: (0, 0)),
        ],
        out_specs=pl.BlockSpec((_ROWS_BLK, HEAD_DIM), lambda i: (i, 0)),
        out_shape=jax.ShapeDtypeStruct((BL, HEAD_DIM), jnp.float32),
    )(rel_ids, att_ids, freq2, phase, table)


def kernel(input_ids_0, input_ids_1, attention_type_ids,
           relative_position_ids, table_0, table_1, attn_type_table,
           inverse_freqs):
    ids0 = input_ids_0.reshape(NW, NCHUNK, CHUNK)
    ids1 = input_ids_1.reshape(NW, NCHUNK, CHUNK)
    emb = _emb_kernel(ids0, ids1, table_0, table_1)

    half = HEAD_DIM // 2
    freq2 = jnp.concatenate([inverse_freqs, inverse_freqs]).reshape(1, HEAD_DIM)
    phase = jnp.concatenate(
        [jnp.zeros((half,), jnp.float32),
         jnp.full((half,), jnp.pi / 2, jnp.float32)]).reshape(1, HEAD_DIM)
    rel = relative_position_ids.reshape(_NBLK, _ROWS_BLK, 1)
    att = attention_type_ids.reshape(_NBLK, _ROWS_BLK, 1)
    rel_att = _pos_kernel(rel, att, freq2, phase, attn_type_table)

    return (emb.reshape(B, L, HIDDEN), rel_att.reshape(B, L, HEAD_DIM))


# speedup vs baseline: 2.0256x; 1.0949x over previous
import functools

import jax
import jax.numpy as jnp
from jax import lax
from jax.experimental import pallas as pl
from jax.experimental.pallas import tpu as pltpu
from jax.experimental.pallas import tpu_sc as plsc

B, L = 1024, 50
HIDDEN = 768
HEAD_DIM = 64
NUM_ATT_TYPES = 8
BL = B * L

NC, NS = 2, 16
NW = NC * NS
ROWS_PER_W = BL // NW
CHUNK = 40
NCHUNK = ROWS_PER_W // CHUNK
LANES = 16
_VECS_PER_ROW = HIDDEN // LANES


def _emb_body(ids0_hbm, ids1_hbm, t0_hbm, t1_hbm, out_hbm,
              idx0_v, idx1_v, bufs0, bufs1, sems0, sems1):
    c = lax.axis_index("c")
    s = lax.axis_index("s")
    wid = s * NC + c
    base = wid * ROWS_PER_W
    pltpu.sync_copy(ids0_hbm.at[pl.ds(base, ROWS_PER_W)], idx0_v)
    pltpu.sync_copy(ids1_hbm.at[pl.ds(base, ROWS_PER_W)], idx1_v)

    def gather_pair(k, slot):
        src0 = t0_hbm.at[idx0_v.at[pl.ds(k * CHUNK, CHUNK)]]
        src1 = t1_hbm.at[idx1_v.at[pl.ds(k * CHUNK, CHUNK)]]
        return (pltpu.make_async_copy(src0, bufs0[slot], sems0[slot]),
                pltpu.make_async_copy(src1, bufs1[slot], sems1[slot]))

    d0, d1 = gather_pair(0, 0)
    d0.start()
    d1.start()

    def outer(g, carry):
        for b in (0, 1):
            k = g * 2 + b

            @pl.when(k + 1 < NCHUNK)
            def _():
                n0, n1 = gather_pair(k + 1, 1 - b)
                n0.start()
                n1.start()

            w0, w1 = gather_pair(k, b)
            w0.wait()
            w1.wait()

            def row_body(r, rc):
                for j in range(_VECS_PER_ROW):
                    plsc.addupdate(bufs0[b].at[r, pl.ds(j * LANES, LANES)],
                                   bufs1[b][r, pl.ds(j * LANES, LANES)])
                return rc

            lax.fori_loop(0, CHUNK, row_body, 0, unroll=False)
            pltpu.sync_copy(bufs0[b],
                            out_hbm.at[pl.ds(base + k * CHUNK, CHUNK)])
        return carry

    lax.fori_loop(0, NCHUNK // 2, outer, 0, unroll=False)


_emb_kernel = functools.partial(
    pl.kernel,
    out_type=jax.ShapeDtypeStruct((BL, HIDDEN), jnp.float32),
    mesh=plsc.VectorSubcoreMesh(core_axis_name="c", subcore_axis_name="s"),
    scratch_types=[
        pltpu.VMEM((ROWS_PER_W,), jnp.int32),
        pltpu.VMEM((ROWS_PER_W,), jnp.int32),
        [pltpu.VMEM((CHUNK, HIDDEN), jnp.float32)] * 2,
        [pltpu.VMEM((CHUNK, HIDDEN), jnp.float32)] * 2,
        [pltpu.SemaphoreType.DMA] * 2,
        [pltpu.SemaphoreType.DMA] * 2,
    ],
)(_emb_body)


_ROWS_BLK = 512
_NBLK = BL // _ROWS_BLK


def _pos_body(rel_ref, att_ref, freq_ref, phase_ref, table_ref, out_ref):
    rel = rel_ref[...].reshape(_ROWS_BLK, 1).astype(jnp.float32)
    x = rel / freq_ref[...] + phase_ref[...]
    posenc = jnp.sin(x)
    att = att_ref[...].reshape(_ROWS_BLK, 1)
    iota = lax.broadcasted_iota(jnp.int32, (_ROWS_BLK, NUM_ATT_TYPES), 1)
    onehot = (att == iota).astype(jnp.float32)
    att_emb = jnp.dot(onehot, table_ref[...],
                      preferred_element_type=jnp.float32)
    out_ref[...] = posenc + att_emb


def _pos_kernel(rel_ids, att_ids, freq2, phase, table):
    return pl.pallas_call(
        _pos_body,
        grid=(_NBLK,),
        in_specs=[
            pl.BlockSpec((1, _ROWS_BLK, 1), lambda i: (i, 0, 0)),
            pl.BlockSpec((1, _ROWS_BLK, 1), lambda i: (i, 0, 0)),
            pl.BlockSpec((1, HEAD_DIM), lambda i: (0, 0)),
            pl.BlockSpec((1, HEAD_DIM), lambda i: (0, 0)),
            pl.BlockSpec((NUM_ATT_TYPES, HEAD_DIM), lambda i: (0, 0)),
        ],
        out_specs=pl.BlockSpec((_ROWS_BLK, HEAD_DIM), lambda i: (i, 0)),
        out_shape=jax.ShapeDtypeStruct((BL, HEAD_DIM), jnp.float32),
    )(rel_ids, att_ids, freq2, phase, table)


def kernel(input_ids_0, input_ids_1, attention_type_ids,
           relative_position_ids, table_0, table_1, attn_type_table,
           inverse_freqs):
    ids0 = input_ids_0.reshape(BL)
    ids1 = input_ids_1.reshape(BL)
    emb = _emb_kernel(ids0, ids1, table_0, table_1)

    half = HEAD_DIM // 2
    freq2 = jnp.concatenate([inverse_freqs, inverse_freqs]).reshape(1, HEAD_DIM)
    phase = jnp.concatenate(
        [jnp.zeros((half,), jnp.float32),
         jnp.full((half,), jnp.pi / 2, jnp.float32)]).reshape(1, HEAD_DIM)
    rel = relative_position_ids.reshape(_NBLK, _ROWS_BLK, 1)
    att = attention_type_ids.reshape(_NBLK, _ROWS_BLK, 1)
    rel_att = _pos_kernel(rel, att, freq2, phase, attn_type_table)

    return (emb.reshape(B, L, HIDDEN), rel_att.reshape(B, L, HEAD_DIM))


# speedup vs baseline: 2.0260x; 1.0002x over previous
import functools

import jax
import jax.numpy as jnp
from jax import lax
from jax.experimental import pallas as pl
from jax.experimental.pallas import tpu as pltpu
from jax.experimental.pallas import tpu_sc as plsc

B, L = 1024, 50
HIDDEN = 768
HEAD_DIM = 64
NUM_ATT_TYPES = 8
BL = B * L

NC, NS = 2, 16
NW = NC * NS
ROWS_PER_W = BL // NW
CHUNK = 40
NCHUNK = ROWS_PER_W // CHUNK
LANES = 16
_VECS_PER_ROW = HIDDEN // LANES


def _emb_body(ids0_hbm, ids1_hbm, t0_hbm, t1_hbm, out_hbm,
              idx0_v, idx1_v, bufs0, bufs1, sems0, sems1):
    c = lax.axis_index("c")
    s = lax.axis_index("s")
    wid = s * NC + c
    base = wid * ROWS_PER_W
    pltpu.sync_copy(ids0_hbm.at[pl.ds(base, ROWS_PER_W)], idx0_v)
    pltpu.sync_copy(ids1_hbm.at[pl.ds(base, ROWS_PER_W)], idx1_v)

    def gather_pair(k, slot):
        src0 = t0_hbm.at[idx0_v.at[pl.ds(k * CHUNK, CHUNK)]]
        src1 = t1_hbm.at[idx1_v.at[pl.ds(k * CHUNK, CHUNK)]]
        return (pltpu.make_async_copy(src0, bufs0[slot], sems0[slot]),
                pltpu.make_async_copy(src1, bufs1[slot], sems1[slot]))

    d0, d1 = gather_pair(0, 0)
    d0.start()
    d1.start()

    def outer(g, carry):
        for b in (0, 1):
            k = g * 2 + b

            @pl.when(k + 1 < NCHUNK)
            def _():
                n0, n1 = gather_pair(k + 1, 1 - b)
                n0.start()
                n1.start()

            w0, w1 = gather_pair(k, b)
            w0.wait()
            w1.wait()

            def row_body(r, rc):
                for j in range(_VECS_PER_ROW):
                    plsc.addupdate(bufs0[b].at[r, pl.ds(j * LANES, LANES)],
                                   bufs1[b][r, pl.ds(j * LANES, LANES)])
                return rc

            lax.fori_loop(0, CHUNK, row_body, 0, unroll=False)
            pltpu.sync_copy(bufs0[b],
                            out_hbm.at[pl.ds(base + k * CHUNK, CHUNK)])
        return carry

    lax.fori_loop(0, NCHUNK // 2, outer, 0, unroll=False)


_emb_kernel = functools.partial(
    pl.kernel,
    out_type=jax.ShapeDtypeStruct((BL, HIDDEN), jnp.float32),
    mesh=plsc.VectorSubcoreMesh(core_axis_name="c", subcore_axis_name="s"),
    compiler_params=pltpu.CompilerParams(use_tc_tiling_on_sc=True),
    scratch_types=[
        pltpu.VMEM((ROWS_PER_W,), jnp.int32),
        pltpu.VMEM((ROWS_PER_W,), jnp.int32),
        [pltpu.VMEM((CHUNK, HIDDEN), jnp.float32)] * 2,
        [pltpu.VMEM((CHUNK, HIDDEN), jnp.float32)] * 2,
        [pltpu.SemaphoreType.DMA] * 2,
        [pltpu.SemaphoreType.DMA] * 2,
    ],
)(_emb_body)


_ROWS_BLK = 512
_NBLK = BL // _ROWS_BLK


def _pos_body(rel_ref, att_ref, freq_ref, phase_ref, table_ref, out_ref):
    rel = rel_ref[...].reshape(_ROWS_BLK, 1).astype(jnp.float32)
    x = rel / freq_ref[...] + phase_ref[...]
    posenc = jnp.sin(x)
    att = att_ref[...].reshape(_ROWS_BLK, 1)
    iota = lax.broadcasted_iota(jnp.int32, (_ROWS_BLK, NUM_ATT_TYPES), 1)
    onehot = (att == iota).astype(jnp.float32)
    att_emb = jnp.dot(onehot, table_ref[...],
                      preferred_element_type=jnp.float32)
    out_ref[...] = posenc + att_emb


def _pos_kernel(rel_ids, att_ids, freq2, phase, table):
    return pl.pallas_call(
        _pos_body,
        grid=(_NBLK,),
        in_specs=[
            pl.BlockSpec((1, _ROWS_BLK, 1), lambda i: (i, 0, 0)),
            pl.BlockSpec((1, _ROWS_BLK, 1), lambda i: (i, 0, 0)),
            pl.BlockSpec((1, HEAD_DIM), lambda i: (0, 0)),
            pl.BlockSpec((1, HEAD_DIM), lambda i: (0, 0)),
            pl.BlockSpec((NUM_ATT_TYPES, HEAD_DIM), lambda i: (0, 0)),
        ],
        out_specs=pl.BlockSpec((_ROWS_BLK, HEAD_DIM), lambda i: (i, 0)),
        out_shape=jax.ShapeDtypeStruct((BL, HEAD_DIM), jnp.float32),
    )(rel_ids, att_ids, freq2, phase, table)


def kernel(input_ids_0, input_ids_1, attention_type_ids,
           relative_position_ids, table_0, table_1, attn_type_table,
           inverse_freqs):
    ids0 = input_ids_0.reshape(BL)
    ids1 = input_ids_1.reshape(BL)
    emb = _emb_kernel(ids0, ids1, table_0, table_1)

    half = HEAD_DIM // 2
    freq2 = jnp.concatenate([inverse_freqs, inverse_freqs]).reshape(1, HEAD_DIM)
    phase = jnp.concatenate(
        [jnp.zeros((half,), jnp.float32),
         jnp.full((half,), jnp.pi / 2, jnp.float32)]).reshape(1, HEAD_DIM)
    rel = relative_position_ids.reshape(_NBLK, _ROWS_BLK, 1)
    att = attention_type_ids.reshape(_NBLK, _ROWS_BLK, 1)
    rel_att = _pos_kernel(rel, att, freq2, phase, attn_type_table)

    return (emb.reshape(B, L, HIDDEN), rel_att.reshape(B, L, HEAD_DIM))
